# Initial kernel scaffold; baseline (speedup 1.0000x reference)
import functools

import jax
import jax.numpy as jnp
from jax import lax
from jax.experimental import pallas as pl
from jax.experimental.pallas import tpu as pltpu
from jax.experimental.pallas import tpu_sc as plsc

NC = 2
NS = 16
NW = NC * NS

_MESH = plsc.VectorSubcoreMesh(
    core_axis_name="c", subcore_axis_name="s", num_cores=NC, num_subcores=NS)


def _sc_degree(dst_resh, ones_col, zeros_col):
    _, nch, ck = dst_resh.shape
    n = zeros_col.shape[0]
    rpt = n // NS

    @functools.partial(
        pl.kernel,
        out_type=jax.ShapeDtypeStruct((NC, n, 1), jnp.float32),
        mesh=_MESH,
        scratch_types=[
            pltpu.VMEM((nch, ck), jnp.int32),
            pltpu.VMEM((ck, 1), jnp.float32),
            pltpu.VMEM_SHARED((n, 1), jnp.float32),
        ],
    )
    def k(dst_hbm, ones_hbm, zeros_hbm, out_hbm, dst_v, ones_v, acc_sh):
        c = lax.axis_index("c")
        s = lax.axis_index("s")
        w = s * NC + c
        pltpu.sync_copy(dst_hbm.at[w], dst_v)
        pltpu.sync_copy(ones_hbm, ones_v)
        r0 = s * rpt
        pltpu.sync_copy(zeros_hbm.at[pl.ds(r0, rpt)], acc_sh.at[pl.ds(r0, rpt)])
        plsc.subcore_barrier()

        @pl.loop(0, nch)
        def _(j):
            pltpu.sync_copy(ones_v, acc_sh.at[dst_v.at[j]], add=True)

        plsc.subcore_barrier()
        pltpu.sync_copy(acc_sh.at[pl.ds(r0, rpt)],
                        out_hbm.at[c, pl.ds(r0, rpt)])

    return k(dst_resh, ones_col, zeros_col)


def _sc_edges(g, src_resh, dst_resh, zeros_nd):
    _, nch, ck = src_resh.shape
    n, d = g.shape
    rpt = n // NS

    @functools.partial(
        pl.kernel,
        out_type=jax.ShapeDtypeStruct((NC, n, d), jnp.float32),
        mesh=_MESH,
        scratch_types=[
            pltpu.VMEM((nch, ck), jnp.int32),
            pltpu.VMEM((nch, ck), jnp.int32),
            pltpu.VMEM((ck, d), jnp.float32),
            pltpu.VMEM((ck, d), jnp.float32),
            pltpu.SemaphoreType.DMA,
            pltpu.SemaphoreType.DMA,
            pltpu.VMEM_SHARED((n, d), jnp.float32),
        ],
    )
    def k(g_hbm, src_hbm, dst_hbm, zeros_hbm, out_hbm,
          src_v, dst_v, rows0, rows1, sem0, sem1, acc_sh):
        c = lax.axis_index("c")
        s = lax.axis_index("s")
        w = s * NC + c
        pltpu.sync_copy(src_hbm.at[w], src_v)
        pltpu.sync_copy(dst_hbm.at[w], dst_v)
        r0 = s * rpt
        pltpu.sync_copy(zeros_hbm.at[pl.ds(r0, rpt)], acc_sh.at[pl.ds(r0, rpt)])
        plsc.subcore_barrier()

        rows = (rows0, rows1)
        sems = (sem0, sem1)
        pltpu.async_copy(g_hbm.at[src_v.at[0]], rows0, sem0)
        pltpu.async_copy(g_hbm.at[src_v.at[1]], rows1, sem1)

        @pl.loop(0, nch, step=2)
        def _(j):
            for b in range(2):
                ch = j + b
                pltpu.make_async_copy(
                    g_hbm.at[pl.ds(0, ck)], rows[b], sems[b]).wait()
                pltpu.sync_copy(rows[b], acc_sh.at[dst_v.at[ch]], add=True)
                nxt = ch + 2

                @pl.when(nxt < nch)
                def _():
                    pltpu.async_copy(g_hbm.at[src_v.at[nxt]], rows[b], sems[b])

        plsc.subcore_barrier()
        pltpu.sync_copy(acc_sh.at[pl.ds(r0, rpt)],
                        out_hbm.at[c, pl.ds(r0, rpt)])

    return k(g, src_resh, dst_resh, zeros_nd)


def _sc_gather_mean(h, nbr_resh, seq_len):
    _, nch, ck = nbr_resh.shape
    n, d = h.shape
    rt = nch * ck
    bt = rt // seq_len
    nv = d // 16

    @functools.partial(
        pl.kernel,
        out_type=(jax.ShapeDtypeStruct((NW * rt, d), jnp.float32),
                  jax.ShapeDtypeStruct((NW * bt, d), jnp.float32)),
        mesh=_MESH,
        scratch_types=[
            pltpu.VMEM((nch, ck), jnp.int32),
            pltpu.VMEM((rt, d), jnp.float32),
            pltpu.VMEM((bt, d), jnp.float32),
            pltpu.SemaphoreType.DMA,
        ],
    )
    def k(h_hbm, nbr_hbm, out_hbm, seq_hbm, nbr_v, rows_v, seq_v, sem):
        c = lax.axis_index("c")
        s = lax.axis_index("s")
        w = s * NC + c
        pltpu.sync_copy(nbr_hbm.at[w], nbr_v)
        for j in range(nch):
            pltpu.async_copy(h_hbm.at[nbr_v.at[j]],
                             rows_v.at[pl.ds(j * ck, ck)], sem)
        pltpu.make_async_copy(h_hbm.at[pl.ds(0, rt)], rows_v, sem).wait()
        pltpu.sync_copy(rows_v, out_hbm.at[pl.ds(w * rt, rt)])

        inv = jnp.float32(1.0 / seq_len)
        for b in range(bt):
            base = b * seq_len

            def body(l, accs):
                return tuple(a + rows_v[base + l, pl.ds(v * 16, 16)]
                             for v, a in enumerate(accs))

            accs = lax.fori_loop(
                0, seq_len, body,
                tuple(jnp.zeros((16,), jnp.float32) for _ in range(nv)))
            for v in range(nv):
                seq_v[b, pl.ds(v * 16, 16)] = accs[v] * inv
        pltpu.sync_copy(seq_v, seq_hbm.at[pl.ds(w * bt, bt)])

    return k(h, nbr_resh)


_TC_R = 1000


def _tc_first_body(ca_ref, cb_ref, emb_ref, w_ref, g_ref, dinv_ref):
    deg = ca_ref[0] + cb_ref[0] + 1.0
    dinv = lax.rsqrt(deg)
    g_ref[...] = dinv * jnp.dot(emb_ref[...], w_ref[...],
                                preferred_element_type=jnp.float32)
    dinv_ref[...] = dinv


def _tc_first(counts, emb, w1):
    n, d = emb.shape
    h = w1.shape[1]
    r = _TC_R
    return pl.pallas_call(
        _tc_first_body,
        grid=(n // r,),
        in_specs=[
            pl.BlockSpec((1, r, 1), lambda i: (0, i, 0)),
            pl.BlockSpec((1, r, 1), lambda i: (1, i, 0)),
            pl.BlockSpec((r, d), lambda i: (i, 0)),
            pl.BlockSpec((d, h), lambda i: (0, 0)),
        ],
        out_specs=[
            pl.BlockSpec((r, h), lambda i: (i, 0)),
            pl.BlockSpec((r, 1), lambda i: (i, 0)),
        ],
        out_shape=[jax.ShapeDtypeStruct((n, h), jnp.float32),
                   jax.ShapeDtypeStruct((n, 1), jnp.float32)],
    )(counts, counts, emb, w1)


def _tc_mid_body(aa_ref, ab_ref, g_ref, dinv_ref, b_ref, w_ref, out_ref):
    x = jnp.maximum(
        dinv_ref[...] * (aa_ref[0] + ab_ref[0] + g_ref[...]) + b_ref[...],
        0.0)
    out_ref[...] = dinv_ref[...] * jnp.dot(x, w_ref[...],
                                           preferred_element_type=jnp.float32)


def _tc_mid(acc, g, dinv, bias, w2):
    n, h = g.shape
    r = _TC_R
    return pl.pallas_call(
        _tc_mid_body,
        grid=(n // r,),
        in_specs=[
            pl.BlockSpec((1, r, h), lambda i: (0, i, 0)),
            pl.BlockSpec((1, r, h), lambda i: (1, i, 0)),
            pl.BlockSpec((r, h), lambda i: (i, 0)),
            pl.BlockSpec((r, 1), lambda i: (i, 0)),
            pl.BlockSpec((1, h), lambda i: (0, 0)),
            pl.BlockSpec((h, h), lambda i: (0, 0)),
        ],
        out_specs=pl.BlockSpec((r, h), lambda i: (i, 0)),
        out_shape=jax.ShapeDtypeStruct((n, h), jnp.float32),
    )(acc, acc, g, dinv, bias, w2)


def _tc_last_body(aa_ref, ab_ref, g_ref, dinv_ref, b_ref, out_ref):
    out_ref[...] = jnp.maximum(
        dinv_ref[...] * (aa_ref[0] + ab_ref[0] + g_ref[...]) + b_ref[...],
        0.0)


def _tc_last(acc, g, dinv, bias):
    n, h = g.shape
    r = _TC_R
    return pl.pallas_call(
        _tc_last_body,
        grid=(n // r,),
        in_specs=[
            pl.BlockSpec((1, r, h), lambda i: (0, i, 0)),
            pl.BlockSpec((1, r, h), lambda i: (1, i, 0)),
            pl.BlockSpec((r, h), lambda i: (i, 0)),
            pl.BlockSpec((r, 1), lambda i: (i, 0)),
            pl.BlockSpec((1, h), lambda i: (0, 0)),
        ],
        out_specs=pl.BlockSpec((r, h), lambda i: (i, 0)),
        out_shape=jax.ShapeDtypeStruct((n, h), jnp.float32),
    )(acc, acc, g, dinv, bias)


def kernel(emb, W1, b1, W2, b2, edge_index, neighbors):
    n, d = emb.shape
    h = W1.shape[1]
    e = edge_index.shape[1]
    bsz, seq_len = neighbors.shape

    ei = edge_index.astype(jnp.int32)
    ep = e // NW
    ck = 100
    nch = ep // ck
    src_resh = ei[0].reshape(NW, nch, ck)
    dst_resh = ei[1].reshape(NW, nch, ck)
    zeros_nd = jnp.zeros((n, d), jnp.float32)
    zeros_col = jnp.zeros((n, 1), jnp.float32)
    ones_col = jnp.ones((ck, 1), jnp.float32)

    counts = _sc_degree(dst_resh, ones_col, zeros_col)
    g1, dinv = _tc_first(counts, emb, W1)
    acc1 = _sc_edges(g1, src_resh, dst_resh, zeros_nd)
    g2 = _tc_mid(acc1, g1, dinv, b1.reshape(1, h), W2)
    acc2 = _sc_edges(g2, src_resh, dst_resh, zeros_nd)
    hfin = _tc_last(acc2, g2, dinv, b2.reshape(1, h))

    bl = bsz * seq_len
    rt = bl // NW
    ck2 = 100
    nbr_resh = neighbors.astype(jnp.int32).reshape(NW, rt // ck2, ck2)
    out_flat, seq_flat = _sc_gather_mean(hfin, nbr_resh, seq_len)
    return (out_flat.reshape(bsz, seq_len, d),
            seq_flat.reshape(bsz, 1, h))


# speedup vs baseline: 17.3362x; 17.3362x over previous
import functools

import jax
import jax.numpy as jnp
from jax import lax
from jax.experimental import pallas as pl
from jax.experimental.pallas import tpu as pltpu
from jax.experimental.pallas import tpu_sc as plsc

NC = 2
NS = 16
NW = NC * NS

_MESH = plsc.VectorSubcoreMesh(
    core_axis_name="c", subcore_axis_name="s", num_cores=NC, num_subcores=NS)


def _tile_copy(src_at, dst_at, s, n):
    chunk = ((n // NS + 7) // 8) * 8
    last = n - (NS - 1) * chunk

    @pl.when(s < NS - 1)
    def _():
        r0 = pl.multiple_of(s * chunk, 8)
        pltpu.sync_copy(src_at(r0, chunk), dst_at(r0, chunk))

    @pl.when(s == NS - 1)
    def _():
        r0 = (NS - 1) * chunk
        pltpu.sync_copy(src_at(r0, last), dst_at(r0, last))


def _sc_degree(dst_resh, ones_col, zeros_col):
    _, nch, ck = dst_resh.shape
    n = zeros_col.shape[0]

    @functools.partial(
        pl.kernel,
        out_type=jax.ShapeDtypeStruct((NC, n, 1), jnp.float32),
        mesh=_MESH,
        scratch_types=[
            pltpu.VMEM((nch, ck), jnp.int32),
            pltpu.VMEM((ck, 1), jnp.float32),
            pltpu.VMEM_SHARED((n, 1), jnp.float32),
        ],
    )
    def k(dst_hbm, ones_hbm, zeros_hbm, out_hbm, dst_v, ones_v, acc_sh):
        c = lax.axis_index("c")
        s = lax.axis_index("s")
        w = s * NC + c
        pltpu.sync_copy(dst_hbm.at[w], dst_v)
        pltpu.sync_copy(ones_hbm, ones_v)
        _tile_copy(lambda r0, sz: zeros_hbm.at[pl.ds(r0, sz)],
                   lambda r0, sz: acc_sh.at[pl.ds(r0, sz)], s, n)
        plsc.subcore_barrier()

        @pl.loop(0, nch)
        def _(j):
            pltpu.sync_copy(ones_v, acc_sh.at[dst_v.at[j]], add=True)

        plsc.subcore_barrier()
        _tile_copy(lambda r0, sz: acc_sh.at[pl.ds(r0, sz)],
                   lambda r0, sz: out_hbm.at[c, pl.ds(r0, sz)], s, n)

    return k(dst_resh, ones_col, zeros_col)


def _sc_edges(gl, gr, src_resh, dst_resh, zeros_nh):
    _, nch, ck = src_resh.shape
    n, dh = gl.shape

    @functools.partial(
        pl.kernel,
        out_type=jax.ShapeDtypeStruct((NC, n, dh), jnp.float32),
        mesh=_MESH,
        scratch_types=[
            pltpu.VMEM((nch, ck), jnp.int32),
            pltpu.VMEM((nch, ck), jnp.int32),
            pltpu.VMEM((ck, dh), jnp.float32),
            pltpu.VMEM((ck, dh), jnp.float32),
            pltpu.SemaphoreType.DMA,
            pltpu.SemaphoreType.DMA,
            pltpu.VMEM_SHARED((n, dh), jnp.float32),
        ],
        compiler_params=pltpu.CompilerParams(use_tc_tiling_on_sc=False),
    )
    def k(gl_hbm, gr_hbm, src_hbm, dst_hbm, zeros_hbm, out_hbm,
          src_v, dst_v, rows0, rows1, sem0, sem1, acc_sh):
        c = lax.axis_index("c")
        s = lax.axis_index("s")
        pltpu.sync_copy(src_hbm.at[s], src_v)
        pltpu.sync_copy(dst_hbm.at[s], dst_v)
        _tile_copy(lambda r0, sz: zeros_hbm.at[pl.ds(r0, sz)],
                   lambda r0, sz: acc_sh.at[pl.ds(r0, sz)], s, n)
        plsc.subcore_barrier()

        rows = (rows0, rows1)
        sems = (sem0, sem1)

        def run(g_hbm):
            pltpu.async_copy(g_hbm.at[src_v.at[0]], rows0, sem0)
            pltpu.async_copy(g_hbm.at[src_v.at[1]], rows1, sem1)

            @pl.loop(0, nch, step=2)
            def _(j):
                for b in range(2):
                    ch = j + b
                    pltpu.make_async_copy(
                        g_hbm.at[src_v.at[ch]], rows[b], sems[b]).wait()
                    pltpu.sync_copy(rows[b], acc_sh.at[dst_v.at[ch]],
                                    add=True)
                    nxt = ch + 2

                    @pl.when(nxt < nch)
                    def _():
                        pltpu.async_copy(g_hbm.at[src_v.at[nxt]],
                                         rows[b], sems[b])

        @pl.when(c == 0)
        def _():
            run(gl_hbm)

        @pl.when(c == 1)
        def _():
            run(gr_hbm)

        plsc.subcore_barrier()
        _tile_copy(lambda r0, sz: acc_sh.at[pl.ds(r0, sz)],
                   lambda r0, sz: out_hbm.at[c, pl.ds(r0, sz)], s, n)

    return k(gl, gr, src_resh, dst_resh, zeros_nh)


def _sc_gather_mean(h, nbr_resh, seq_len):
    _, nch, ck = nbr_resh.shape
    n, d = h.shape
    rt = nch * ck
    bt = rt // seq_len
    nv = d // 16

    @functools.partial(
        pl.kernel,
        out_type=(jax.ShapeDtypeStruct((NW * rt, d), jnp.float32),
                  jax.ShapeDtypeStruct((NW * bt, d), jnp.float32)),
        mesh=_MESH,
        scratch_types=[
            pltpu.VMEM((nch, ck), jnp.int32),
            pltpu.VMEM((rt, d), jnp.float32),
            pltpu.VMEM((bt, d), jnp.float32),
            pltpu.SemaphoreType.DMA,
        ],
    )
    def k(h_hbm, nbr_hbm, out_hbm, seq_hbm, nbr_v, rows_v, seq_v, sem):
        c = lax.axis_index("c")
        s = lax.axis_index("s")
        w = s * NC + c
        pltpu.sync_copy(nbr_hbm.at[w], nbr_v)
        for j in range(nch):
            pltpu.async_copy(h_hbm.at[nbr_v.at[j]],
                             rows_v.at[pl.ds(j * ck, ck)], sem)
        pltpu.make_async_copy(h_hbm.at[pl.ds(0, rt)], rows_v, sem).wait()
        pltpu.sync_copy(rows_v, out_hbm.at[pl.ds(pl.multiple_of(w * rt, 8),
                                                 rt)])

        inv = jnp.float32(1.0 / seq_len)
        for b in range(bt):
            base = b * seq_len

            def body(l, accs):
                return tuple(a + rows_v[base + l, pl.ds(v * 16, 16)]
                             for v, a in enumerate(accs))

            accs = lax.fori_loop(
                0, seq_len, body,
                tuple(jnp.zeros((16,), jnp.float32) for _ in range(nv)))
            for v in range(nv):
                seq_v[b, pl.ds(v * 16, 16)] = accs[v] * inv
        pltpu.sync_copy(seq_v, seq_hbm.at[pl.ds(pl.multiple_of(w * bt, 8),
                                                bt)])

    return k(h, nbr_resh)


_TC_R = 1000


def _tc_first_body(ca_ref, cb_ref, emb_ref, w_ref, gl_ref, gr_ref, dinv_ref):
    deg = ca_ref[0] + cb_ref[0] + 1.0
    dinv = lax.rsqrt(deg)
    g = dinv * jnp.dot(emb_ref[...], w_ref[...],
                       preferred_element_type=jnp.float32)
    dh = g.shape[1] // 2
    gl_ref[...] = g[:, :dh]
    gr_ref[...] = g[:, dh:]
    dinv_ref[...] = dinv


def _tc_first(counts, emb, w1):
    n, d = emb.shape
    h = w1.shape[1]
    r = _TC_R
    return pl.pallas_call(
        _tc_first_body,
        grid=(n // r,),
        in_specs=[
            pl.BlockSpec((1, r, 1), lambda i: (0, i, 0)),
            pl.BlockSpec((1, r, 1), lambda i: (1, i, 0)),
            pl.BlockSpec((r, d), lambda i: (i, 0)),
            pl.BlockSpec((d, h), lambda i: (0, 0)),
        ],
        out_specs=[
            pl.BlockSpec((r, h // 2), lambda i: (i, 0)),
            pl.BlockSpec((r, h // 2), lambda i: (i, 0)),
            pl.BlockSpec((r, 1), lambda i: (i, 0)),
        ],
        out_shape=[jax.ShapeDtypeStruct((n, h // 2), jnp.float32),
                   jax.ShapeDtypeStruct((n, h // 2), jnp.float32),
                   jax.ShapeDtypeStruct((n, 1), jnp.float32)],
    )(counts, counts, emb, w1)


def _tc_mid_body(aa_ref, ab_ref, gl_ref, gr_ref, dinv_ref, b_ref, w_ref,
                 ol_ref, or_ref):
    agg = jnp.concatenate([aa_ref[0] + gl_ref[...], ab_ref[0] + gr_ref[...]],
                          axis=1)
    x = jnp.maximum(dinv_ref[...] * agg + b_ref[...], 0.0)
    g = dinv_ref[...] * jnp.dot(x, w_ref[...],
                                preferred_element_type=jnp.float32)
    dh = g.shape[1] // 2
    ol_ref[...] = g[:, :dh]
    or_ref[...] = g[:, dh:]


def _tc_mid(acc, gl, gr, dinv, bias, w2):
    n, dh = gl.shape
    h = 2 * dh
    r = _TC_R
    return pl.pallas_call(
        _tc_mid_body,
        grid=(n // r,),
        in_specs=[
            pl.BlockSpec((1, r, dh), lambda i: (0, i, 0)),
            pl.BlockSpec((1, r, dh), lambda i: (1, i, 0)),
            pl.BlockSpec((r, dh), lambda i: (i, 0)),
            pl.BlockSpec((r, dh), lambda i: (i, 0)),
            pl.BlockSpec((r, 1), lambda i: (i, 0)),
            pl.BlockSpec((1, h), lambda i: (0, 0)),
            pl.BlockSpec((h, h), lambda i: (0, 0)),
        ],
        out_specs=[
            pl.BlockSpec((r, dh), lambda i: (i, 0)),
            pl.BlockSpec((r, dh), lambda i: (i, 0)),
        ],
        out_shape=[jax.ShapeDtypeStruct((n, dh), jnp.float32),
                   jax.ShapeDtypeStruct((n, dh), jnp.float32)],
    )(acc, acc, gl, gr, dinv, bias, w2)


def _tc_last_body(aa_ref, ab_ref, gl_ref, gr_ref, dinv_ref, b_ref, out_ref):
    agg = jnp.concatenate([aa_ref[0] + gl_ref[...], ab_ref[0] + gr_ref[...]],
                          axis=1)
    out_ref[...] = jnp.maximum(dinv_ref[...] * agg + b_ref[...], 0.0)


def _tc_last(acc, gl, gr, dinv, bias):
    n, dh = gl.shape
    h = 2 * dh
    r = _TC_R
    return pl.pallas_call(
        _tc_last_body,
        grid=(n // r,),
        in_specs=[
            pl.BlockSpec((1, r, dh), lambda i: (0, i, 0)),
            pl.BlockSpec((1, r, dh), lambda i: (1, i, 0)),
            pl.BlockSpec((r, dh), lambda i: (i, 0)),
            pl.BlockSpec((r, dh), lambda i: (i, 0)),
            pl.BlockSpec((r, 1), lambda i: (i, 0)),
            pl.BlockSpec((1, h), lambda i: (0, 0)),
        ],
        out_specs=pl.BlockSpec((r, h), lambda i: (i, 0)),
        out_shape=jax.ShapeDtypeStruct((n, h), jnp.float32),
    )(acc, acc, gl, gr, dinv, bias)


def kernel(emb, W1, b1, W2, b2, edge_index, neighbors):
    n, d = emb.shape
    h = W1.shape[1]
    e = edge_index.shape[1]
    bsz, seq_len = neighbors.shape

    ei = edge_index.astype(jnp.int32)
    ck = 50
    nch32 = e // (NW * ck)
    dst32 = ei[1].reshape(NW, nch32, ck)
    nch16 = e // (NS * ck)
    src16 = ei[0].reshape(NS, nch16, ck)
    dst16 = ei[1].reshape(NS, nch16, ck)

    zeros_nh = jnp.zeros((n, h // 2), jnp.float32)
    zeros_col = jnp.zeros((n, 1), jnp.float32)
    ones_col = jnp.ones((ck, 1), jnp.float32)

    counts = _sc_degree(dst32, ones_col, zeros_col)
    gl1, gr1, dinv = _tc_first(counts, emb, W1)
    acc1 = _sc_edges(gl1, gr1, src16, dst16, zeros_nh)
    gl2, gr2 = _tc_mid(acc1, gl1, gr1, dinv, b1.reshape(1, h), W2)
    acc2 = _sc_edges(gl2, gr2, src16, dst16, zeros_nh)
    hfin = _tc_last(acc2, gl2, gr2, dinv, b2.reshape(1, h))

    bl = bsz * seq_len
    rt = bl // NW
    ck2 = 100
    nbr_resh = neighbors.astype(jnp.int32).reshape(NW, rt // ck2, ck2)
    out_flat, seq_flat = _sc_gather_mean(hfin, nbr_resh, seq_len)
    return (out_flat.reshape(bsz, seq_len, d),
            seq_flat.reshape(bsz, 1, h))


# speedup vs baseline: 20.0012x; 1.1537x over previous
import functools

import jax
import jax.numpy as jnp
from jax import lax
from jax.experimental import pallas as pl
from jax.experimental.pallas import tpu as pltpu
from jax.experimental.pallas import tpu_sc as plsc

NC = 2
NS = 16
NW = NC * NS

_MESH = plsc.VectorSubcoreMesh(
    core_axis_name="c", subcore_axis_name="s", num_cores=NC, num_subcores=NS)


def _tile_copy(src_at, dst_at, s, n):
    chunk = ((n // NS + 7) // 8) * 8
    last = n - (NS - 1) * chunk

    @pl.when(s < NS - 1)
    def _():
        r0 = pl.multiple_of(s * chunk, 8)
        pltpu.sync_copy(src_at(r0, chunk), dst_at(r0, chunk))

    @pl.when(s == NS - 1)
    def _():
        r0 = (NS - 1) * chunk
        pltpu.sync_copy(src_at(r0, last), dst_at(r0, last))


def _sc_degree(dst_resh, ones_col, zeros_col):
    _, nch, ck = dst_resh.shape
    n = zeros_col.shape[0]

    @functools.partial(
        pl.kernel,
        out_type=jax.ShapeDtypeStruct((NC, n, 1), jnp.float32),
        mesh=_MESH,
        scratch_types=[
            pltpu.VMEM((nch, ck), jnp.int32),
            pltpu.VMEM((ck, 1), jnp.float32),
            pltpu.VMEM_SHARED((n, 1), jnp.float32),
        ],
    )
    def k(dst_hbm, ones_hbm, zeros_hbm, out_hbm, dst_v, ones_v, acc_sh):
        c = lax.axis_index("c")
        s = lax.axis_index("s")
        w = s * NC + c
        pltpu.sync_copy(dst_hbm.at[w], dst_v)
        pltpu.sync_copy(ones_hbm, ones_v)
        _tile_copy(lambda r0, sz: zeros_hbm.at[pl.ds(r0, sz)],
                   lambda r0, sz: acc_sh.at[pl.ds(r0, sz)], s, n)
        plsc.subcore_barrier()

        @pl.loop(0, nch)
        def _(j):
            pltpu.sync_copy(ones_v, acc_sh.at[dst_v.at[j]], add=True)

        plsc.subcore_barrier()
        _tile_copy(lambda r0, sz: acc_sh.at[pl.ds(r0, sz)],
                   lambda r0, sz: out_hbm.at[c, pl.ds(r0, sz)], s, n)

    return k(dst_resh, ones_col, zeros_col)


def _sc_edges(g, src_resh, dst_resh, zeros_nd, n_chunks):
    _, cpt, ck = src_resh.shape
    n, d = g.shape

    @functools.partial(
        pl.kernel,
        out_type=jax.ShapeDtypeStruct((NC, n, d), jnp.float32),
        mesh=_MESH,
        scratch_types=[
            pltpu.VMEM((cpt, ck), jnp.int32),
            pltpu.VMEM((cpt, ck), jnp.int32),
            pltpu.VMEM((ck, d), jnp.float32),
            pltpu.SemaphoreType.DMA,
            pltpu.VMEM_SHARED((n, d), jnp.float32),
        ],
    )
    def k(g_hbm, src_hbm, dst_hbm, zeros_hbm, out_hbm,
          src_v, dst_v, rows_v, sem, acc_sh):
        c = lax.axis_index("c")
        s = lax.axis_index("s")
        w = s * NC + c
        pltpu.sync_copy(src_hbm.at[w], src_v)
        pltpu.sync_copy(dst_hbm.at[w], dst_v)
        _tile_copy(lambda r0, sz: zeros_hbm.at[pl.ds(r0, sz)],
                   lambda r0, sz: acc_sh.at[pl.ds(r0, sz)], s, n)
        plsc.subcore_barrier()

        nch_eff = jnp.clip(n_chunks - w * cpt, 0, cpt)

        @pl.when(nch_eff > 0)
        def _():
            pltpu.async_copy(g_hbm.at[src_v.at[0]], rows_v, sem)

        @pl.loop(0, nch_eff)
        def _(ch):
            pltpu.make_async_copy(
                g_hbm.at[src_v.at[ch]], rows_v, sem).wait()
            pltpu.sync_copy(rows_v, acc_sh.at[dst_v.at[ch]], add=True)

            @pl.when(ch + 1 < nch_eff)
            def _():
                pltpu.async_copy(g_hbm.at[src_v.at[ch + 1]], rows_v, sem)

        plsc.subcore_barrier()
        _tile_copy(lambda r0, sz: acc_sh.at[pl.ds(r0, sz)],
                   lambda r0, sz: out_hbm.at[c, pl.ds(r0, sz)], s, n)

    return k(g, src_resh, dst_resh, zeros_nd)


def _sc_gather_mean(h, nbr_resh, seq_len):
    _, nch, ck = nbr_resh.shape
    n, d = h.shape
    rt = nch * ck
    bt = rt // seq_len
    nv = d // 16

    @functools.partial(
        pl.kernel,
        out_type=(jax.ShapeDtypeStruct((NW * rt, d), jnp.float32),
                  jax.ShapeDtypeStruct((NW * bt, d), jnp.float32)),
        mesh=_MESH,
        scratch_types=[
            pltpu.VMEM((nch, ck), jnp.int32),
            pltpu.VMEM((rt, d), jnp.float32),
            pltpu.VMEM((bt, d), jnp.float32),
            pltpu.SemaphoreType.DMA,
        ],
    )
    def k(h_hbm, nbr_hbm, out_hbm, seq_hbm, nbr_v, rows_v, seq_v, sem):
        c = lax.axis_index("c")
        s = lax.axis_index("s")
        w = s * NC + c
        pltpu.sync_copy(nbr_hbm.at[w], nbr_v)
        for j in range(nch):
            pltpu.async_copy(h_hbm.at[nbr_v.at[j]],
                             rows_v.at[pl.ds(j * ck, ck)], sem)
        pltpu.make_async_copy(h_hbm.at[pl.ds(0, rt)], rows_v, sem).wait()
        pltpu.sync_copy(rows_v, out_hbm.at[pl.ds(pl.multiple_of(w * rt, 8),
                                                 rt)])

        inv = jnp.float32(1.0 / seq_len)
        for b in range(bt):
            base = b * seq_len

            def body(l, accs):
                return tuple(a + rows_v[base + l, pl.ds(v * 16, 16)]
                             for v, a in enumerate(accs))

            accs = lax.fori_loop(
                0, seq_len, body,
                tuple(jnp.zeros((16,), jnp.float32) for _ in range(nv)))
            for v in range(nv):
                seq_v[b, pl.ds(v * 16, 16)] = accs[v] * inv
        pltpu.sync_copy(seq_v, seq_hbm.at[pl.ds(pl.multiple_of(w * bt, 8),
                                                bt)])

    return k(h, nbr_resh)


_TC_R = 1000


def _tc_first_body(ca_ref, cb_ref, emb_ref, w_ref, g_ref, dinv_ref):
    deg = ca_ref[0] + cb_ref[0] + 1.0
    dinv = lax.rsqrt(deg)
    g_ref[...] = dinv * jnp.dot(emb_ref[...], w_ref[...],
                                preferred_element_type=jnp.float32)
    dinv_ref[...] = dinv


def _tc_first(counts, emb, w1):
    n, d = emb.shape
    h = w1.shape[1]
    r = _TC_R
    return pl.pallas_call(
        _tc_first_body,
        grid=(n // r,),
        in_specs=[
            pl.BlockSpec((1, r, 1), lambda i: (0, i, 0)),
            pl.BlockSpec((1, r, 1), lambda i: (1, i, 0)),
            pl.BlockSpec((r, d), lambda i: (i, 0)),
            pl.BlockSpec((d, h), lambda i: (0, 0)),
        ],
        out_specs=[
            pl.BlockSpec((r, h), lambda i: (i, 0)),
            pl.BlockSpec((r, 1), lambda i: (i, 0)),
        ],
        out_shape=[jax.ShapeDtypeStruct((n, h), jnp.float32),
                   jax.ShapeDtypeStruct((n, 1), jnp.float32)],
    )(counts, counts, emb, w1)


def _tc_mid_body(aa_ref, ab_ref, g_ref, dinv_ref, b_ref, w_ref, out_ref):
    agg = aa_ref[0] + ab_ref[0] + g_ref[...]
    x = jnp.maximum(dinv_ref[...] * agg + b_ref[...], 0.0)
    out_ref[...] = dinv_ref[...] * jnp.dot(x, w_ref[...],
                                           preferred_element_type=jnp.float32)


def _tc_mid(acc, g, dinv, bias, w2):
    n, h = g.shape
    r = _TC_R
    return pl.pallas_call(
        _tc_mid_body,
        grid=(n // r,),
        in_specs=[
            pl.BlockSpec((1, r, h), lambda i: (0, i, 0)),
            pl.BlockSpec((1, r, h), lambda i: (1, i, 0)),
            pl.BlockSpec((r, h), lambda i: (i, 0)),
            pl.BlockSpec((r, 1), lambda i: (i, 0)),
            pl.BlockSpec((1, h), lambda i: (0, 0)),
            pl.BlockSpec((h, h), lambda i: (0, 0)),
        ],
        out_specs=pl.BlockSpec((r, h), lambda i: (i, 0)),
        out_shape=jax.ShapeDtypeStruct((n, h), jnp.float32),
    )(acc, acc, g, dinv, bias, w2)


def _tc_last_body(aa_ref, ab_ref, g_ref, dinv_ref, b_ref, out_ref):
    agg = aa_ref[0] + ab_ref[0] + g_ref[...]
    out_ref[...] = jnp.maximum(dinv_ref[...] * agg + b_ref[...], 0.0)


def _tc_last(acc, g, dinv, bias):
    n, h = g.shape
    r = _TC_R
    return pl.pallas_call(
        _tc_last_body,
        grid=(n // r,),
        in_specs=[
            pl.BlockSpec((1, r, h), lambda i: (0, i, 0)),
            pl.BlockSpec((1, r, h), lambda i: (1, i, 0)),
            pl.BlockSpec((r, h), lambda i: (i, 0)),
            pl.BlockSpec((r, 1), lambda i: (i, 0)),
            pl.BlockSpec((1, h), lambda i: (0, 0)),
        ],
        out_specs=pl.BlockSpec((r, h), lambda i: (i, 0)),
        out_shape=jax.ShapeDtypeStruct((n, h), jnp.float32),
    )(acc, acc, g, dinv, bias)


def kernel(emb, W1, b1, W2, b2, edge_index, neighbors):
    n, d = emb.shape
    h = W1.shape[1]
    e = edge_index.shape[1]
    bsz, seq_len = neighbors.shape

    ei = edge_index.astype(jnp.int32)
    ckd = 50
    nchd = e // (NW * ckd)
    dst_deg = ei[1].reshape(NW, nchd, ckd)
    ck = 128
    n_chunks = e // ck
    cpt = -(-n_chunks // NW)
    pad = NW * cpt * ck - e
    eip = jnp.pad(ei, ((0, 0), (0, pad)))
    src_resh = eip[0].reshape(NW, cpt, ck)
    dst_resh = eip[1].reshape(NW, cpt, ck)

    zeros_nd = jnp.zeros((n, d), jnp.float32)
    zeros_col = jnp.zeros((n, 1), jnp.float32)
    ones_col = jnp.ones((ckd, 1), jnp.float32)

    counts = _sc_degree(dst_deg, ones_col, zeros_col)
    g1, dinv = _tc_first(counts, emb, W1)
    acc1 = _sc_edges(g1, src_resh, dst_resh, zeros_nd, n_chunks)
    g2 = _tc_mid(acc1, g1, dinv, b1.reshape(1, h), W2)
    acc2 = _sc_edges(g2, src_resh, dst_resh, zeros_nd, n_chunks)
    hfin = _tc_last(acc2, g2, dinv, b2.reshape(1, h))

    bl = bsz * seq_len
    rt = bl // NW
    ck2 = 100
    nbr_resh = neighbors.astype(jnp.int32).reshape(NW, rt // ck2, ck2)
    out_flat, seq_flat = _sc_gather_mean(hfin, nbr_resh, seq_len)
    return (out_flat.reshape(bsz, seq_len, d),
            seq_flat.reshape(bsz, 1, h))


# speedup vs baseline: 26.2870x; 1.3143x over previous
import functools

import jax
import jax.numpy as jnp
from jax import lax
from jax.experimental import pallas as pl
from jax.experimental.pallas import tpu as pltpu
from jax.experimental.pallas import tpu_sc as plsc

NC = 2
NS = 16
NW = NC * NS

_MESH = plsc.VectorSubcoreMesh(
    core_axis_name="c", subcore_axis_name="s", num_cores=NC, num_subcores=NS)


def _tile_copy(src_at, dst_at, s, n):
    chunk = ((n // NS + 7) // 8) * 8
    last = n - (NS - 1) * chunk

    @pl.when(s < NS - 1)
    def _():
        r0 = pl.multiple_of(s * chunk, 8)
        pltpu.sync_copy(src_at(r0, chunk), dst_at(r0, chunk))

    @pl.when(s == NS - 1)
    def _():
        r0 = (NS - 1) * chunk
        pltpu.sync_copy(src_at(r0, last), dst_at(r0, last))


def _sc_edges(g, packed_idx, zeros_nd, cpt, n_chunks, sc_tiling=False):
    _, ck = packed_idx.shape
    n, d = g.shape
    nv = ck // 16

    @functools.partial(
        pl.kernel,
        out_type=jax.ShapeDtypeStruct((NC, n, d), jnp.float32),
        mesh=_MESH,
        scratch_types=[
            pltpu.VMEM((cpt, ck), jnp.int32),
            pltpu.VMEM((2, ck), jnp.int32),
            pltpu.VMEM((2, ck), jnp.int32),
            pltpu.VMEM((ck, d), jnp.float32),
            pltpu.VMEM((ck, d), jnp.float32),
            pltpu.SemaphoreType.DMA,
            pltpu.SemaphoreType.DMA,
            pltpu.VMEM_SHARED((n, d), jnp.float32),
        ],
        compiler_params=(pltpu.CompilerParams(use_tc_tiling_on_sc=False)
                         if sc_tiling else None),
    )
    def k(g_hbm, idx_hbm, zeros_hbm, out_hbm,
          idx_v, src_st, dst_st, rows0, rows1, sem0, sem1, acc_sh):
        c = lax.axis_index("c")
        s = lax.axis_index("s")
        w = s * NC + c
        pltpu.sync_copy(idx_hbm.at[pl.ds(pl.multiple_of(w * cpt, 8), cpt)],
                        idx_v)
        _tile_copy(lambda r0, sz: zeros_hbm.at[pl.ds(r0, sz)],
                   lambda r0, sz: acc_sh.at[pl.ds(r0, sz)], s, n)
        plsc.subcore_barrier()

        nch_eff = jnp.clip(n_chunks - w * cpt, 0, cpt)
        rows = (rows0, rows1)
        sems = (sem0, sem1)

        def unpack(ch, b):
            for v in range(nv):
                sl = pl.ds(v * 16, 16)
                p = idx_v[ch, sl]
                src_st[b, sl] = p & 0xFFFF
                dst_st[b, sl] = lax.shift_right_logical(p, 16)

        def start_gather(b):
            pltpu.async_copy(g_hbm.at[src_st.at[b]], rows[b], sems[b])

        for b in range(2):
            @pl.when(b < nch_eff)
            def _():
                unpack(b, b)
                start_gather(b)

        @pl.loop(0, cpt, step=2)
        def _(j):
            for b in range(2):
                ch = j + b

                @pl.when(ch < nch_eff)
                def _():
                    pltpu.make_async_copy(
                        g_hbm.at[src_st.at[b]], rows[b], sems[b]).wait()
                    pltpu.sync_copy(rows[b], acc_sh.at[dst_st.at[b]],
                                    add=True)

                    @pl.when(ch + 2 < nch_eff)
                    def _():
                        unpack(ch + 2, b)
                        start_gather(b)

        plsc.subcore_barrier()
        _tile_copy(lambda r0, sz: acc_sh.at[pl.ds(r0, sz)],
                   lambda r0, sz: out_hbm.at[c, pl.ds(r0, sz)], s, n)

    return k(g, packed_idx, zeros_nd)


def _sc_gather_mean(h, nbr_resh, seq_len):
    _, nch, ck = nbr_resh.shape
    n, d = h.shape
    rt = nch * ck
    bt = rt // seq_len
    nv = d // 16

    @functools.partial(
        pl.kernel,
        out_type=(jax.ShapeDtypeStruct((NW * rt, d), jnp.float32),
                  jax.ShapeDtypeStruct((NW * bt, d), jnp.float32)),
        mesh=_MESH,
        scratch_types=[
            pltpu.VMEM((nch, ck), jnp.int32),
            pltpu.VMEM((rt, d), jnp.float32),
            pltpu.VMEM((bt, d), jnp.float32),
            pltpu.SemaphoreType.DMA,
        ],
    )
    def k(h_hbm, nbr_hbm, out_hbm, seq_hbm, nbr_v, rows_v, seq_v, sem):
        c = lax.axis_index("c")
        s = lax.axis_index("s")
        w = s * NC + c
        pltpu.sync_copy(nbr_hbm.at[w], nbr_v)
        for j in range(nch):
            pltpu.async_copy(h_hbm.at[nbr_v.at[j]],
                             rows_v.at[pl.ds(j * ck, ck)], sem)
        pltpu.make_async_copy(h_hbm.at[pl.ds(0, rt)], rows_v, sem).wait()
        pltpu.sync_copy(rows_v, out_hbm.at[pl.ds(pl.multiple_of(w * rt, 8),
                                                 rt)])

        inv = jnp.float32(1.0 / seq_len)
        for b in range(bt):
            base = b * seq_len

            def body(l, accs):
                return tuple(a + rows_v[base + l, pl.ds(v * 16, 16)]
                             for v, a in enumerate(accs))

            accs = lax.fori_loop(
                0, seq_len, body,
                tuple(jnp.zeros((16,), jnp.float32) for _ in range(nv)))
            for v in range(nv):
                seq_v[b, pl.ds(v * 16, 16)] = accs[v] * inv
        pltpu.sync_copy(seq_v, seq_hbm.at[pl.ds(pl.multiple_of(w * bt, 8),
                                                bt)])

    return k(h, nbr_resh)


_TC_R = 1000


def _tc_first_body(ca_ref, cb_ref, emb_ref, w_ref, g_ref, dinv_ref):
    deg = ca_ref[0][:, :1] + cb_ref[0][:, :1] + 1.0
    dinv = lax.rsqrt(deg)
    g_ref[...] = dinv * jnp.dot(emb_ref[...], w_ref[...],
                                preferred_element_type=jnp.float32)
    dinv_ref[...] = dinv


def _tc_first(counts, emb, w1):
    n, d = emb.shape
    h = w1.shape[1]
    r = _TC_R
    return pl.pallas_call(
        _tc_first_body,
        grid=(n // r,),
        in_specs=[
            pl.BlockSpec((1, r, counts.shape[2]), lambda i: (0, i, 0)),
            pl.BlockSpec((1, r, counts.shape[2]), lambda i: (1, i, 0)),
            pl.BlockSpec((r, d), lambda i: (i, 0)),
            pl.BlockSpec((d, h), lambda i: (0, 0)),
        ],
        out_specs=[
            pl.BlockSpec((r, h), lambda i: (i, 0)),
            pl.BlockSpec((r, 1), lambda i: (i, 0)),
        ],
        out_shape=[jax.ShapeDtypeStruct((n, h), jnp.float32),
                   jax.ShapeDtypeStruct((n, 1), jnp.float32)],
    )(counts, counts, emb, w1)


def _tc_mid_body(aa_ref, ab_ref, g_ref, dinv_ref, b_ref, w_ref, out_ref):
    agg = aa_ref[0] + ab_ref[0] + g_ref[...]
    x = jnp.maximum(dinv_ref[...] * agg + b_ref[...], 0.0)
    out_ref[...] = dinv_ref[...] * jnp.dot(x, w_ref[...],
                                           preferred_element_type=jnp.float32)


def _tc_mid(acc, g, dinv, bias, w2):
    n, h = g.shape
    r = _TC_R
    return pl.pallas_call(
        _tc_mid_body,
        grid=(n // r,),
        in_specs=[
            pl.BlockSpec((1, r, h), lambda i: (0, i, 0)),
            pl.BlockSpec((1, r, h), lambda i: (1, i, 0)),
            pl.BlockSpec((r, h), lambda i: (i, 0)),
            pl.BlockSpec((r, 1), lambda i: (i, 0)),
            pl.BlockSpec((1, h), lambda i: (0, 0)),
            pl.BlockSpec((h, h), lambda i: (0, 0)),
        ],
        out_specs=pl.BlockSpec((r, h), lambda i: (i, 0)),
        out_shape=jax.ShapeDtypeStruct((n, h), jnp.float32),
    )(acc, acc, g, dinv, bias, w2)


def _tc_last_body(aa_ref, ab_ref, g_ref, dinv_ref, b_ref, out_ref):
    agg = aa_ref[0] + ab_ref[0] + g_ref[...]
    out_ref[...] = jnp.maximum(dinv_ref[...] * agg + b_ref[...], 0.0)


def _tc_last(acc, g, dinv, bias):
    n, h = g.shape
    r = _TC_R
    return pl.pallas_call(
        _tc_last_body,
        grid=(n // r,),
        in_specs=[
            pl.BlockSpec((1, r, h), lambda i: (0, i, 0)),
            pl.BlockSpec((1, r, h), lambda i: (1, i, 0)),
            pl.BlockSpec((r, h), lambda i: (i, 0)),
            pl.BlockSpec((r, 1), lambda i: (i, 0)),
            pl.BlockSpec((1, h), lambda i: (0, 0)),
        ],
        out_specs=pl.BlockSpec((r, h), lambda i: (i, 0)),
        out_shape=jax.ShapeDtypeStruct((n, h), jnp.float32),
    )(acc, acc, g, dinv, bias)


def kernel(emb, W1, b1, W2, b2, edge_index, neighbors):
    n, d = emb.shape
    h = W1.shape[1]
    e = edge_index.shape[1]
    bsz, seq_len = neighbors.shape

    ei = edge_index.astype(jnp.int32)
    ck = 128
    n_chunks = e // ck
    cpt = ((-(-n_chunks // NW) + 7) // 8) * 8
    pad = NW * cpt * ck - e
    packed = jnp.pad(ei[0] | (ei[1] << 16), (0, pad)).reshape(NW * cpt, ck)

    zeros_nd = jnp.zeros((n, d), jnp.float32)

    counts = _sc_edges(jnp.ones((n, 16), jnp.float32), packed,
                       jnp.zeros((n, 16), jnp.float32), cpt, n_chunks,
                       sc_tiling=True)
    g1, dinv = _tc_first(counts, emb, W1)
    acc1 = _sc_edges(g1, packed, zeros_nd, cpt, n_chunks)
    g2 = _tc_mid(acc1, g1, dinv, b1.reshape(1, h), W2)
    acc2 = _sc_edges(g2, packed, zeros_nd, cpt, n_chunks)
    hfin = _tc_last(acc2, g2, dinv, b2.reshape(1, h))

    bl = bsz * seq_len
    rt = bl // NW
    ck2 = 100
    nbr_resh = neighbors.astype(jnp.int32).reshape(NW, rt // ck2, ck2)
    out_flat, seq_flat = _sc_gather_mean(hfin, nbr_resh, seq_len)
    return (out_flat.reshape(bsz, seq_len, d),
            seq_flat.reshape(bsz, 1, h))


# speedup vs baseline: 26.5579x; 1.0103x over previous
import functools

import jax
import jax.numpy as jnp
from jax import lax
from jax.experimental import pallas as pl
from jax.experimental.pallas import tpu as pltpu
from jax.experimental.pallas import tpu_sc as plsc

NC = 2
NS = 16
NW = NC * NS

_MESH = plsc.VectorSubcoreMesh(
    core_axis_name="c", subcore_axis_name="s", num_cores=NC, num_subcores=NS)


def _tile_copy(src_at, dst_at, s, n):
    chunk = ((n // NS + 7) // 8) * 8
    last = n - (NS - 1) * chunk

    @pl.when(s < NS - 1)
    def _():
        r0 = pl.multiple_of(s * chunk, 8)
        pltpu.sync_copy(src_at(r0, chunk), dst_at(r0, chunk))

    @pl.when(s == NS - 1)
    def _():
        r0 = (NS - 1) * chunk
        pltpu.sync_copy(src_at(r0, last), dst_at(r0, last))


def _sc_edges(g, packed_idx, zeros_nd, cpt, n_chunks, sc_tiling=False):
    _, ck = packed_idx.shape
    n, d = g.shape
    nv = ck // 16

    @functools.partial(
        pl.kernel,
        out_type=jax.ShapeDtypeStruct((NC, n, d), jnp.float32),
        mesh=_MESH,
        scratch_types=[
            pltpu.VMEM((cpt, ck), jnp.int32),
            pltpu.VMEM((2, ck), jnp.int32),
            pltpu.VMEM((2, ck), jnp.int32),
            pltpu.VMEM((ck, d), jnp.float32),
            pltpu.VMEM((ck, d), jnp.float32),
            pltpu.SemaphoreType.DMA,
            pltpu.SemaphoreType.DMA,
            pltpu.VMEM_SHARED((n, d), jnp.float32),
        ],
        compiler_params=(pltpu.CompilerParams(use_tc_tiling_on_sc=False)
                         if sc_tiling else None),
    )
    def k(g_hbm, idx_hbm, zeros_hbm, out_hbm,
          idx_v, src_st, dst_st, rows0, rows1, sem0, sem1, acc_sh):
        c = lax.axis_index("c")
        s = lax.axis_index("s")
        w = s * NC + c
        pltpu.sync_copy(idx_hbm.at[pl.ds(pl.multiple_of(w * cpt, 8), cpt)],
                        idx_v)
        _tile_copy(lambda r0, sz: zeros_hbm.at[pl.ds(r0, sz)],
                   lambda r0, sz: acc_sh.at[pl.ds(r0, sz)], s, n)
        plsc.subcore_barrier()

        nch_eff = jnp.clip(n_chunks - w * cpt, 0, cpt)
        rows = (rows0, rows1)
        sems = (sem0, sem1)

        def unpack(ch, b):
            for v in range(nv):
                sl = pl.ds(v * 16, 16)
                p = idx_v[ch, sl]
                src_st[b, sl] = p & 0xFFFF
                dst_st[b, sl] = lax.shift_right_logical(p, 16)

        def start_gather(b):
            pltpu.async_copy(g_hbm.at[src_st.at[b]], rows[b], sems[b])

        for b in range(2):
            @pl.when(b < nch_eff)
            def _():
                unpack(b, b)
                start_gather(b)

        @pl.loop(0, cpt, step=2)
        def _(j):
            for b in range(2):
                ch = j + b

                @pl.when(ch < nch_eff)
                def _():
                    pltpu.make_async_copy(
                        g_hbm.at[src_st.at[b]], rows[b], sems[b]).wait()
                    pltpu.sync_copy(rows[b], acc_sh.at[dst_st.at[b]],
                                    add=True)

                    @pl.when(ch + 2 < nch_eff)
                    def _():
                        unpack(ch + 2, b)
                        start_gather(b)

        plsc.subcore_barrier()
        _tile_copy(lambda r0, sz: acc_sh.at[pl.ds(r0, sz)],
                   lambda r0, sz: out_hbm.at[c, pl.ds(r0, sz)], s, n)

    return k(g, packed_idx, zeros_nd)


def _sc_hist(ones_nd, dst_idx, zeros_nd, cpt, n_chunks):
    _, ck = dst_idx.shape
    n, d = ones_nd.shape

    @functools.partial(
        pl.kernel,
        out_type=jax.ShapeDtypeStruct((NC, n, d), jnp.float32),
        mesh=_MESH,
        scratch_types=[
            pltpu.VMEM((cpt, ck), jnp.int32),
            pltpu.VMEM((ck, d), jnp.float32),
            pltpu.VMEM((ck, d), jnp.float32),
            pltpu.SemaphoreType.DMA,
            pltpu.SemaphoreType.DMA,
            pltpu.VMEM_SHARED((n, d), jnp.float32),
        ],
        compiler_params=pltpu.CompilerParams(use_tc_tiling_on_sc=False),
    )
    def k(g_hbm, idx_hbm, zeros_hbm, out_hbm,
          idx_v, rows0, rows1, sem0, sem1, acc_sh):
        c = lax.axis_index("c")
        s = lax.axis_index("s")
        w = s * NC + c
        pltpu.sync_copy(idx_hbm.at[pl.ds(pl.multiple_of(w * cpt, 8), cpt)],
                        idx_v)
        _tile_copy(lambda r0, sz: zeros_hbm.at[pl.ds(r0, sz)],
                   lambda r0, sz: acc_sh.at[pl.ds(r0, sz)], s, n)
        plsc.subcore_barrier()

        nch_eff = jnp.clip(n_chunks - w * cpt, 0, cpt)
        rows = (rows0, rows1)
        sems = (sem0, sem1)

        for b in range(2):
            @pl.when(b < nch_eff)
            def _():
                pltpu.async_copy(g_hbm.at[idx_v.at[b]], rows[b], sems[b])

        @pl.loop(0, cpt, step=2)
        def _(j):
            for b in range(2):
                ch = j + b

                @pl.when(ch < nch_eff)
                def _():
                    pltpu.make_async_copy(
                        g_hbm.at[idx_v.at[ch]], rows[b], sems[b]).wait()
                    pltpu.sync_copy(rows[b], acc_sh.at[idx_v.at[ch]],
                                    add=True)

                    @pl.when(ch + 2 < nch_eff)
                    def _():
                        pltpu.async_copy(g_hbm.at[idx_v.at[ch + 2]],
                                         rows[b], sems[b])

        plsc.subcore_barrier()
        _tile_copy(lambda r0, sz: acc_sh.at[pl.ds(r0, sz)],
                   lambda r0, sz: out_hbm.at[c, pl.ds(r0, sz)], s, n)

    return k(ones_nd, dst_idx, zeros_nd)


def _sc_gather_mean(h, nbr_resh, seq_len):
    _, nch, ck = nbr_resh.shape
    n, d = h.shape
    rt = nch * ck
    bt = rt // seq_len
    nv = d // 16

    @functools.partial(
        pl.kernel,
        out_type=(jax.ShapeDtypeStruct((NW * rt, d), jnp.float32),
                  jax.ShapeDtypeStruct((NW * bt, d), jnp.float32)),
        mesh=_MESH,
        scratch_types=[
            pltpu.VMEM((nch, ck), jnp.int32),
            pltpu.VMEM((rt, d), jnp.float32),
            pltpu.VMEM((bt, d), jnp.float32),
            pltpu.SemaphoreType.DMA,
        ],
    )
    def k(h_hbm, nbr_hbm, out_hbm, seq_hbm, nbr_v, rows_v, seq_v, sem):
        c = lax.axis_index("c")
        s = lax.axis_index("s")
        w = s * NC + c
        pltpu.sync_copy(nbr_hbm.at[w], nbr_v)
        for j in range(nch):
            pltpu.async_copy(h_hbm.at[nbr_v.at[j]],
                             rows_v.at[pl.ds(j * ck, ck)], sem)
        pltpu.make_async_copy(h_hbm.at[pl.ds(0, rt)], rows_v, sem).wait()
        pltpu.sync_copy(rows_v, out_hbm.at[pl.ds(pl.multiple_of(w * rt, 8),
                                                 rt)])

        inv = jnp.float32(1.0 / seq_len)
        for b in range(bt):
            base = b * seq_len

            def body(l, accs):
                return tuple(a + rows_v[base + l, pl.ds(v * 16, 16)]
                             for v, a in enumerate(accs))

            accs = lax.fori_loop(
                0, seq_len, body,
                tuple(jnp.zeros((16,), jnp.float32) for _ in range(nv)))
            for v in range(nv):
                seq_v[b, pl.ds(v * 16, 16)] = accs[v] * inv
        pltpu.sync_copy(seq_v, seq_hbm.at[pl.ds(pl.multiple_of(w * bt, 8),
                                                bt)])

    return k(h, nbr_resh)


_TC_R = 1000


def _tc_first_body(ca_ref, cb_ref, emb_ref, w_ref, g_ref, dinv_ref):
    deg = ca_ref[0][:, :1] + cb_ref[0][:, :1] + 1.0
    dinv = lax.rsqrt(deg)
    g_ref[...] = dinv * jnp.dot(emb_ref[...], w_ref[...],
                                preferred_element_type=jnp.float32)
    dinv_ref[...] = dinv


def _tc_first(counts, emb, w1):
    n, d = emb.shape
    h = w1.shape[1]
    r = _TC_R
    return pl.pallas_call(
        _tc_first_body,
        grid=(n // r,),
        in_specs=[
            pl.BlockSpec((1, r, counts.shape[2]), lambda i: (0, i, 0)),
            pl.BlockSpec((1, r, counts.shape[2]), lambda i: (1, i, 0)),
            pl.BlockSpec((r, d), lambda i: (i, 0)),
            pl.BlockSpec((d, h), lambda i: (0, 0)),
        ],
        out_specs=[
            pl.BlockSpec((r, h), lambda i: (i, 0)),
            pl.BlockSpec((r, 1), lambda i: (i, 0)),
        ],
        out_shape=[jax.ShapeDtypeStruct((n, h), jnp.float32),
                   jax.ShapeDtypeStruct((n, 1), jnp.float32)],
    )(counts, counts, emb, w1)


def _tc_mid_body(aa_ref, ab_ref, g_ref, dinv_ref, b_ref, w_ref, out_ref):
    agg = aa_ref[0] + ab_ref[0] + g_ref[...]
    x = jnp.maximum(dinv_ref[...] * agg + b_ref[...], 0.0)
    out_ref[...] = dinv_ref[...] * jnp.dot(x, w_ref[...],
                                           preferred_element_type=jnp.float32)


def _tc_mid(acc, g, dinv, bias, w2):
    n, h = g.shape
    r = _TC_R
    return pl.pallas_call(
        _tc_mid_body,
        grid=(n // r,),
        in_specs=[
            pl.BlockSpec((1, r, h), lambda i: (0, i, 0)),
            pl.BlockSpec((1, r, h), lambda i: (1, i, 0)),
            pl.BlockSpec((r, h), lambda i: (i, 0)),
            pl.BlockSpec((r, 1), lambda i: (i, 0)),
            pl.BlockSpec((1, h), lambda i: (0, 0)),
            pl.BlockSpec((h, h), lambda i: (0, 0)),
        ],
        out_specs=pl.BlockSpec((r, h), lambda i: (i, 0)),
        out_shape=jax.ShapeDtypeStruct((n, h), jnp.float32),
    )(acc, acc, g, dinv, bias, w2)


def _tc_last_body(aa_ref, ab_ref, g_ref, dinv_ref, b_ref, out_ref):
    agg = aa_ref[0] + ab_ref[0] + g_ref[...]
    out_ref[...] = jnp.maximum(dinv_ref[...] * agg + b_ref[...], 0.0)


def _tc_last(acc, g, dinv, bias):
    n, h = g.shape
    r = _TC_R
    return pl.pallas_call(
        _tc_last_body,
        grid=(n // r,),
        in_specs=[
            pl.BlockSpec((1, r, h), lambda i: (0, i, 0)),
            pl.BlockSpec((1, r, h), lambda i: (1, i, 0)),
            pl.BlockSpec((r, h), lambda i: (i, 0)),
            pl.BlockSpec((r, 1), lambda i: (i, 0)),
            pl.BlockSpec((1, h), lambda i: (0, 0)),
        ],
        out_specs=pl.BlockSpec((r, h), lambda i: (i, 0)),
        out_shape=jax.ShapeDtypeStruct((n, h), jnp.float32),
    )(acc, acc, g, dinv, bias)


def kernel(emb, W1, b1, W2, b2, edge_index, neighbors):
    n, d = emb.shape
    h = W1.shape[1]
    e = edge_index.shape[1]
    bsz, seq_len = neighbors.shape

    ei = edge_index.astype(jnp.int32)
    ck = 128
    n_chunks = e // ck
    cpt = ((-(-n_chunks // NW) + 7) // 8) * 8
    pad = NW * cpt * ck - e
    packed = jnp.pad(ei[0] | (ei[1] << 16), (0, pad)).reshape(NW * cpt, ck)

    zeros_nd = jnp.zeros((n, d), jnp.float32)

    dst_idx = jnp.pad(ei[1], (0, pad)).reshape(NW * cpt, ck)
    counts = _sc_hist(jnp.ones((n, 16), jnp.float32), dst_idx,
                      jnp.zeros((n, 16), jnp.float32), cpt, n_chunks)
    g1, dinv = _tc_first(counts, emb, W1)
    acc1 = _sc_edges(g1, packed, zeros_nd, cpt, n_chunks)
    g2 = _tc_mid(acc1, g1, dinv, b1.reshape(1, h), W2)
    acc2 = _sc_edges(g2, packed, zeros_nd, cpt, n_chunks)
    hfin = _tc_last(acc2, g2, dinv, b2.reshape(1, h))

    bl = bsz * seq_len
    rt = bl // NW
    ck2 = 100
    nbr_resh = neighbors.astype(jnp.int32).reshape(NW, rt // ck2, ck2)
    out_flat, seq_flat = _sc_gather_mean(hfin, nbr_resh, seq_len)
    return (out_flat.reshape(bsz, seq_len, d),
            seq_flat.reshape(bsz, 1, h))
